# baseline (speedup 1.0000x reference)
import functools

import jax
import jax.numpy as jnp
from jax import lax
from jax.experimental import pallas as pl
from jax.experimental.pallas import tpu as pltpu
from jax.experimental.pallas import tpu_sc as plsc

_CH = 128


@jax.jit
def _lookup(idx2d, weight):
    ncht, ch = idx2d.shape
    _, d = weight.shape
    info = plsc.get_sparse_core_info()
    ncores = info.num_cores
    nw = ncores * info.num_subcores
    nch = ncht // nw
    rounds = nch // 2
    n = ncht * ch

    mesh = plsc.VectorSubcoreMesh(core_axis_name="c", subcore_axis_name="s")

    @functools.partial(
        pl.kernel,
        out_type=jax.ShapeDtypeStruct((n, d), jnp.float32),
        mesh=mesh,
        scratch_types=[
            pltpu.VMEM((nch, ch), jnp.int32),
            pltpu.VMEM((ch, d), jnp.float32),
            pltpu.VMEM((ch, d), jnp.float32),
            pltpu.SemaphoreType.DMA,
            pltpu.SemaphoreType.DMA,
        ],
        compiler_params=pltpu.CompilerParams(use_tc_tiling_on_sc=False),
    )
    def emb(idx_hbm, tab_hbm, out_hbm, idx_v, buf0, buf1, sem0, sem1):
        bufs = (buf0, buf1)
        sems = (sem0, sem1)
        wid = lax.axis_index("s") * ncores + lax.axis_index("c")
        c0 = wid * nch
        row0 = c0 * ch
        pltpu.sync_copy(idx_hbm.at[pl.ds(c0, nch)], idx_v)

        def g_start(j, b):
            pltpu.async_copy(tab_hbm.at[idx_v.at[j]], bufs[b], sems[b])

        def g_wait(j, b):
            pltpu.make_async_copy(tab_hbm.at[idx_v.at[j]], bufs[b], sems[b]).wait()

        g_start(0, 0)
        g_start(1, 1)

        def body(i, carry):
            for b in range(2):
                j = 2 * i + b
                g_wait(j, b)
                pltpu.sync_copy(bufs[b], out_hbm.at[pl.ds(row0 + j * ch, ch)])
                g_start(j + 2, b)
            return carry

        lax.fori_loop(0, rounds - 1, body, 0)
        for b in range(2):
            j = 2 * (rounds - 1) + b
            g_wait(j, b)
            pltpu.sync_copy(bufs[b], out_hbm.at[pl.ds(row0 + j * ch, ch)])

    return emb(idx2d, weight)


def kernel(x, weight):
    batch, fields = x.shape
    _, d = weight.shape
    n = batch * fields
    idx2d = x.reshape(n // _CH, _CH).astype(jnp.int32)
    out = _lookup(idx2d, weight)
    return out.reshape(batch, fields, d)
